# baseline (speedup 1.0000x reference)
import jax
import jax.numpy as jnp
from jax import lax
from jax.experimental import pallas as pl
from jax.experimental.pallas import tpu as pltpu


def _fused_kernel(vec_ref, rows_ref, vvec_ref, wvn_ref, wnode_ref,
                  wnp_ref, hvec_ref, vvec_new_ref, sums, counts, u):
    p = pl.program_id(0)
    i = pl.program_id(1)
    G = sums.shape[0]
    rows = rows_ref[...].reshape(1, -1)
    B = rows.shape[1]
    iota = lax.broadcasted_iota(jnp.int32, (G, B), 0)
    onehot_t = (iota == rows).astype(jnp.bfloat16)

    @pl.when((p == 0) & (i == 0))
    def _():
        sums[...] = jnp.zeros_like(sums)
        counts[...] = jnp.zeros_like(counts)

    @pl.when(p == 0)
    def _():
        blk = vec_ref[...].astype(jnp.bfloat16)
        sums[...] += lax.dot_general(
            onehot_t, blk, (((1,), (0,)), ((), ())),
            preferred_element_type=jnp.float32)
        counts[...] += jnp.sum(
            onehot_t.astype(jnp.float32), axis=1, keepdims=True)

    @pl.when((p == 1) & (i == 0))
    def _():
        mean = sums[...] / jnp.maximum(counts[...], 1.0)
        vvec_new = lax.dot_general(
            mean + vvec_ref[...], wvn_ref[...], (((1,), (1,)), ((), ())),
            preferred_element_type=jnp.float32)
        vvec_new_ref[...] = vvec_new
        u[...] = lax.dot_general(
            vvec_new, wnode_ref[...], (((1,), (1,)), ((), ())),
            preferred_element_type=jnp.float32).astype(jnp.bfloat16)

    @pl.when(p == 1)
    def _():
        blk = vec_ref[...].astype(jnp.bfloat16)
        prod = lax.dot_general(
            blk, wnp_ref[...], (((1,), (0,)), ((), ())),
            preferred_element_type=jnp.float32)
        gath = lax.dot_general(
            onehot_t, u[...], (((0,), (0,)), ((), ())),
            preferred_element_type=jnp.float32)
        hvec_ref[...] = prod + gath


def kernel(vec, batch, vvec, W_vn, W_node):
    N, D = vec.shape
    G = vvec.shape[0]
    B = 5000
    if N % B != 0:
        for cand in (4000, 2000, 1000, 1024, 800, 512, 250, 125, 100, 8, 1):
            if N % cand == 0:
                B = cand
                break
    nb = N // B
    rows3 = batch.astype(jnp.int32).reshape(nb, 1, B)
    wnp = (W_node.T + jnp.eye(D, dtype=jnp.float32)).astype(jnp.bfloat16)

    hvec, vvec_new = pl.pallas_call(
        _fused_kernel,
        grid=(2, nb),
        in_specs=[
            pl.BlockSpec((B, D), lambda p, i: (i, 0)),
            pl.BlockSpec((1, 1, B), lambda p, i: (i, 0, 0)),
            pl.BlockSpec((G, D), lambda p, i: (0, 0)),
            pl.BlockSpec((D, D), lambda p, i: (0, 0)),
            pl.BlockSpec((D, D), lambda p, i: (0, 0)),
            pl.BlockSpec((D, D), lambda p, i: (0, 0)),
        ],
        out_specs=[
            pl.BlockSpec((B, D), lambda p, i: (jnp.where(p == 0, 0, i), 0)),
            pl.BlockSpec((G, D), lambda p, i: (0, 0)),
        ],
        out_shape=[
            jax.ShapeDtypeStruct((N, D), jnp.float32),
            jax.ShapeDtypeStruct((G, D), jnp.float32),
        ],
        scratch_shapes=[
            pltpu.VMEM((G, D), jnp.float32),
            pltpu.VMEM((G, 1), jnp.float32),
            pltpu.VMEM((G, D), jnp.bfloat16),
        ],
        compiler_params=pltpu.CompilerParams(
            dimension_semantics=("arbitrary", "arbitrary")),
    )(vec, rows3, vvec, W_vn, W_node, wnp)

    return (hvec, vvec_new)


# speedup vs baseline: 1.0231x; 1.0231x over previous
import jax
import jax.numpy as jnp
from jax import lax
from jax.experimental import pallas as pl
from jax.experimental.pallas import tpu as pltpu


def _fused_kernel(vec_ref, rows_ref, vvec_ref, wvn_ref, wnode_ref,
                  hvec_ref, vvec_new_ref, sums, counts, u):
    p = pl.program_id(0)
    i = pl.program_id(1)
    G = sums.shape[0]
    rows = rows_ref[...].reshape(1, -1)
    B = rows.shape[1]
    iota = lax.broadcasted_iota(jnp.int32, (G, B), 0)
    onehot_t = (iota == rows).astype(jnp.bfloat16)

    @pl.when((p == 0) & (i == 0))
    def _():
        sums[...] = jnp.zeros_like(sums)
        counts[...] = jnp.zeros_like(counts)

    @pl.when(p == 0)
    def _():
        blk = vec_ref[...].astype(jnp.bfloat16)
        sums[...] += lax.dot_general(
            onehot_t, blk, (((1,), (0,)), ((), ())),
            preferred_element_type=jnp.float32)
        counts[...] += jnp.sum(
            onehot_t.astype(jnp.float32), axis=1, keepdims=True)

    @pl.when((p == 1) & (i == 0))
    def _():
        mean = sums[...] / jnp.maximum(counts[...], 1.0)
        vvec_new = lax.dot_general(
            mean + vvec_ref[...], wvn_ref[...], (((1,), (1,)), ((), ())),
            preferred_element_type=jnp.float32)
        vvec_new_ref[...] = vvec_new
        u[...] = lax.dot_general(
            vvec_new, wnode_ref[...], (((1,), (1,)), ((), ())),
            preferred_element_type=jnp.float32).astype(jnp.bfloat16)

    @pl.when(p == 1)
    def _():
        blk = vec_ref[...]
        prod = lax.dot_general(
            blk.astype(jnp.bfloat16), wnode_ref[...].astype(jnp.bfloat16),
            (((1,), (1,)), ((), ())),
            preferred_element_type=jnp.float32)
        gath = lax.dot_general(
            onehot_t, u[...], (((0,), (0,)), ((), ())),
            preferred_element_type=jnp.float32)
        hvec_ref[...] = blk + prod + gath


def kernel(vec, batch, vvec, W_vn, W_node):
    N, D = vec.shape
    G = vvec.shape[0]
    B = 5000
    if N % B != 0:
        for cand in (4000, 2000, 1000, 1024, 800, 512, 250, 125, 100, 8, 1):
            if N % cand == 0:
                B = cand
                break
    nb = N // B
    rows3 = batch.astype(jnp.int32).reshape(nb, 1, B)

    hvec, vvec_new = pl.pallas_call(
        _fused_kernel,
        grid=(2, nb),
        in_specs=[
            pl.BlockSpec(
                (B, D),
                lambda p, i: (jnp.where(p == 0, nb - 1 - i, i), 0)),
            pl.BlockSpec(
                (1, 1, B),
                lambda p, i: (jnp.where(p == 0, nb - 1 - i, i), 0, 0)),
            pl.BlockSpec((G, D), lambda p, i: (0, 0)),
            pl.BlockSpec((D, D), lambda p, i: (0, 0)),
            pl.BlockSpec((D, D), lambda p, i: (0, 0)),
        ],
        out_specs=[
            pl.BlockSpec((B, D), lambda p, i: (jnp.where(p == 0, 0, i), 0)),
            pl.BlockSpec((G, D), lambda p, i: (0, 0)),
        ],
        out_shape=[
            jax.ShapeDtypeStruct((N, D), jnp.float32),
            jax.ShapeDtypeStruct((G, D), jnp.float32),
        ],
        scratch_shapes=[
            pltpu.VMEM((G, D), jnp.float32),
            pltpu.VMEM((G, 1), jnp.float32),
            pltpu.VMEM((G, D), jnp.bfloat16),
        ],
        compiler_params=pltpu.CompilerParams(
            dimension_semantics=("arbitrary", "arbitrary")),
    )(vec, rows3, vvec, W_vn, W_node)

    return (hvec, vvec_new)
